# Initial kernel scaffold; baseline (speedup 1.0000x reference)
import jax
import jax.numpy as jnp
from jax.experimental import pallas as pl


def kernel(source_nodes, target_nodes, edge_features, current_time, raw_memory, all_prototypes, We, be, tw, tb, Wq, bq, Wg, bg, temperature, ln_g, ln_b, pln_g, pln_b):
    raise NotImplementedError("write your pallas kernel here")


# speedup vs baseline: 1.4017x; 1.4017x over previous
import functools

import jax
import jax.numpy as jnp
from jax import lax
from jax.experimental import pallas as pl
from jax.experimental.pallas import tpu as pltpu


def _ln(x, g, b):
    m = jnp.mean(x, axis=-1, keepdims=True)
    v = jnp.mean((x - m) ** 2, axis=-1, keepdims=True)
    return (x - m) / jnp.sqrt(v + 1e-5) * g + b


def _dense_body(edge_ref, time_ref, gmem_ref, gproto_ref, WeT_ref, be_ref,
                tw_ref, tb_ref, WqmT_ref, WqeT_ref, WqtT_ref, bq_ref,
                wgm_ref, wgc_ref, wgt_ref, bg_ref, temp_ref,
                lng_ref, lnb_ref, plng_ref, plnb_ref, u_ref):
    K = gproto_ref.shape[-1] // gmem_ref.shape[-1]
    D = gmem_ref.shape[-1]

    t = time_ref[...]
    te = jnp.cos(t * tw_ref[...] + tb_ref[...])

    ep = lax.dot(edge_ref[...], WeT_ref[...],
                 precision=lax.Precision.HIGHEST,
                 preferred_element_type=jnp.float32) + be_ref[...]
    nrm = jnp.sqrt(jnp.sum(ep * ep, axis=-1, keepdims=True))
    ep = ep / (nrm + 1e-8) * 10.0
    ep = jnp.clip(ep, -10.0, 10.0)

    ep_q = lax.dot(ep, WqeT_ref[...], precision=lax.Precision.HIGHEST,
                   preferred_element_type=jnp.float32)
    te_q = lax.dot(te, WqtT_ref[...], precision=lax.Precision.HIGHEST,
                   preferred_element_type=jnp.float32)
    te_g = jnp.sum(te * wgt_ref[...], axis=-1, keepdims=True)

    lng, lnb = lng_ref[...], lnb_ref[...]
    temp = jnp.clip(temp_ref[0, 0], 0.05, 2.0) + 1e-6

    for side in range(2):
        mem = gmem_ref[side]
        proto = gproto_ref[side]

        qi = (lax.dot(mem, WqmT_ref[...], precision=lax.Precision.HIGHEST,
                      preferred_element_type=jnp.float32)
              + ep_q + te_q + bq_ref[...])
        q = jnp.tanh(_ln(qi, lng, lnb))
        qn = q / jnp.maximum(jnp.sqrt(jnp.sum(q * q, axis=-1, keepdims=True)),
                             1e-12)

        sims = []
        pks = []
        for k in range(K):
            pk = _ln(proto[:, k * D:(k + 1) * D], plng_ref[...], plnb_ref[...])
            pks.append(pk)
            pn = pk / jnp.maximum(
                jnp.sqrt(jnp.sum(pk * pk, axis=-1, keepdims=True)), 1e-12)
            sims.append(jnp.sum(qn * pn, axis=-1, keepdims=True))
        sim = jnp.concatenate(sims, axis=-1) / temp
        sim = sim - jnp.max(sim, axis=-1, keepdims=True)
        e = jnp.exp(sim)
        attn = e / jnp.sum(e, axis=-1, keepdims=True)

        cand = attn[:, 0:1] * pks[0]
        for k in range(1, K):
            cand = cand + attn[:, k:k + 1] * pks[k]
        cand = jnp.clip(cand, -5.0, 5.0)

        g = (jnp.sum(jnp.clip(mem, -100.0, 100.0) * wgm_ref[...],
                     axis=-1, keepdims=True)
             + jnp.sum(cand * wgc_ref[...], axis=-1, keepdims=True)
             + te_g + bg_ref[0, 0])
        gate = 1.0 / (1.0 + jnp.exp(-g))

        upd = (1.0 - gate) * mem + gate * cand
        u_ref[side] = jnp.clip(_ln(upd, lng, lnb), -50.0, 50.0)


def _dense_phase(edge_features, time2d, gmem, gproto, WeT, be, tw, tb,
                 WqmT, WqeT, WqtT, bq, wgm, wgc, wgt, bg, temperature,
                 ln_g, ln_b, pln_g, pln_b, R=512):
    Bp = edge_features.shape[0]
    D = gmem.shape[-1]
    KD = gproto.shape[-1]
    grid = Bp // R
    full = lambda shape: pl.BlockSpec(shape, lambda i: (0,) * len(shape))
    return pl.pallas_call(
        _dense_body,
        grid=(grid,),
        in_specs=[
            pl.BlockSpec((R, edge_features.shape[1]), lambda i: (i, 0)),
            pl.BlockSpec((R, 1), lambda i: (i, 0)),
            pl.BlockSpec((2, R, D), lambda i: (0, i, 0)),
            pl.BlockSpec((2, R, KD), lambda i: (0, i, 0)),
            full((64, 128)), full((1, 128)), full((1, 64)), full((1, 64)),
            full((128, 128)), full((128, 128)), full((64, 128)),
            full((1, 128)), full((1, 128)), full((1, 128)), full((1, 64)),
            full((1, 1)), full((1, 1)),
            full((1, 128)), full((1, 128)), full((1, 128)), full((1, 128)),
        ],
        out_specs=pl.BlockSpec((2, R, D), lambda i: (0, i, 0)),
        out_shape=jax.ShapeDtypeStruct((2, Bp, D), jnp.float32),
    )(edge_features, time2d, gmem, gproto, WeT, be, tw, tb,
      WqmT, WqeT, WqtT, bq, wgm, wgc, wgt, bg, temperature,
      ln_g, ln_b, pln_g, pln_b)


def kernel(source_nodes, target_nodes, edge_features, current_time,
           raw_memory, all_prototypes, We, be, tw, tb, Wq, bq, Wg, bg,
           temperature, ln_g, ln_b, pln_g, pln_b):
    N, D = raw_memory.shape
    B = source_nodes.shape[0]
    K = all_prototypes.shape[1]
    TD = tw.shape[0]

    c_idx = jnp.concatenate([source_nodes, target_nodes]).astype(jnp.int32)
    order = jnp.arange(2 * B, dtype=jnp.int32)
    ticket = jnp.zeros((N,), jnp.int32).at[c_idx].max(order)
    jgather = ticket[c_idx]

    gmem = raw_memory[c_idx].reshape(2, B, D)
    gproto = all_prototypes.reshape(N, K * D)[c_idx].reshape(2, B, K * D)

    WeT = We.T
    WqmT = Wq[:, :D].T
    WqeT = Wq[:, D:2 * D].T
    WqtT = Wq[:, 2 * D:].T
    wgm = Wg[0, :D].reshape(1, D)
    wgc = Wg[0, D:2 * D].reshape(1, D)
    wgt = Wg[0, 2 * D:].reshape(1, TD)

    u = _dense_phase(
        edge_features, current_time.reshape(B, 1), gmem, gproto,
        WeT, be.reshape(1, D), tw.reshape(1, TD), tb.reshape(1, TD),
        WqmT, WqeT, WqtT, bq.reshape(1, D),
        wgm, wgc, wgt, bg.reshape(1, 1), temperature.reshape(1, 1),
        ln_g.reshape(1, D), ln_b.reshape(1, D),
        pln_g.reshape(1, D), pln_b.reshape(1, D))

    u_flat = u.reshape(2 * B, D)

    out = raw_memory.at[c_idx].set(u_flat[jgather])
    return out
